# baseline (speedup 1.0000x reference)
import functools

import jax
import jax.numpy as jnp
from jax import lax
from jax.experimental import pallas as pl
from jax.experimental.pallas import tpu as pltpu
from jax.experimental.pallas import tpu_sc as plsc

_NEG = -1e30
_T = 256
_TT = 1024


def _gate_body(xg_ref, nz_ref, wg_ref, wn_ref,
               idx_ref, w_ref, gr_ref, seg_ref, gmap_ref, cnt_ref):
    t = pl.program_id(0)
    nt = pl.num_programs(0) - 1
    n_exp = wg_ref.shape[0]

    @pl.when(t == 0)
    def _init():
        cnt_ref[...] = jnp.zeros_like(cnt_ref)

    @pl.when(t < nt)
    def _tile():
        xg = xg_ref[...]
        dn = (((1,), (1,)), ((), ()))
        clean = jax.lax.dot_general(xg, wg_ref[...], dn,
                                    preferred_element_type=jnp.float32)
        raw = jax.lax.dot_general(xg, wn_ref[...], dn,
                                  preferred_element_type=jnp.float32)
        sp = jnp.maximum(raw, 0.0) + jnp.log1p(jnp.exp(-jnp.abs(raw)))
        noisy = clean + nz_ref[...] * sp
        cols = jax.lax.broadcasted_iota(jnp.int32, noisy.shape, 1)
        m1 = jnp.max(noisy, axis=1, keepdims=True)
        i1 = jnp.min(jnp.where(noisy == m1, cols, n_exp), axis=1,
                     keepdims=True)
        oh1 = (cols == i1).astype(jnp.float32)
        masked = jnp.where(cols == i1, _NEG, noisy)
        m2 = jnp.max(masked, axis=1, keepdims=True)
        i2 = jnp.min(jnp.where(masked == m2, cols, n_exp), axis=1,
                     keepdims=True)
        oh2 = (cols == i2).astype(jnp.float32)
        z = jnp.exp(m2 - m1)
        w1 = 1.0 / (1.0 + z)
        w2 = z / (1.0 + z)

        idx_ref[...] = jnp.concatenate(
            [i1.astype(jnp.int32), i2.astype(jnp.int32)], axis=1)
        w_ref[...] = jnp.concatenate([w1, w2], axis=1)

        ohs = oh1 + oh2
        r = jax.lax.broadcasted_iota(jnp.int32, (ohs.shape[0],) * 2, 0)
        c = jax.lax.broadcasted_iota(jnp.int32, (ohs.shape[0],) * 2, 1)
        tril = (r > c).astype(jnp.float32)
        prior = jax.lax.dot_general(tril, ohs, (((1,), (0,)), ((), ())),
                                    preferred_element_type=jnp.float32)
        carry = cnt_ref[...]
        g1 = jnp.sum((prior + carry) * oh1, axis=1, keepdims=True)
        g2 = jnp.sum((prior + carry) * oh2, axis=1, keepdims=True)
        gr_ref[...] = jnp.concatenate(
            [g1.astype(jnp.int32), g2.astype(jnp.int32)], axis=1)
        cnt_ref[...] = carry + jnp.sum(ohs, axis=0, keepdims=True)

    @pl.when(t == nt)
    def _finalize():
        counts = cnt_ref[...]
        seg_size = jnp.ceil(counts / _T) * _T
        r = jax.lax.broadcasted_iota(jnp.int32, (n_exp, n_exp), 0)
        c = jax.lax.broadcasted_iota(jnp.int32, (n_exp, n_exp), 1)
        excl = (r < c).astype(jnp.float32)
        seg_start = jax.lax.dot_general(
            seg_size, excl, (((1,), (0,)), ((), ())),
            preferred_element_type=jnp.float32)
        ends = seg_start + seg_size
        seg_ref[...] = seg_start.astype(jnp.int32)
        n_pad = gmap_ref.shape[1]
        jt = jax.lax.broadcasted_iota(
            jnp.int32, (n_pad, n_exp), 0).astype(jnp.float32) * _T
        gm = jnp.sum((jt >= ends).astype(jnp.int32), axis=1)
        gm = jnp.minimum(gm, n_exp - 1)
        n_used = (jnp.sum(seg_size) / _T).astype(jnp.int32)
        jcol = jax.lax.broadcasted_iota(jnp.int32, (n_pad,), 0)
        gm = jnp.where(jcol == n_pad - 8, n_used, gm)
        gmap_ref[...] = gm.reshape(1, n_pad)


def _gating(x_gate, noise, Wg, Wn, n_pad):
    n, d = x_gate.shape
    e = Wg.shape[0]
    nt = n // _TT
    last = nt - 1
    return pl.pallas_call(
        _gate_body,
        grid=(nt + 1,),
        in_specs=[
            pl.BlockSpec((_TT, d), lambda t: (jnp.minimum(t, last), 0)),
            pl.BlockSpec((_TT, e), lambda t: (jnp.minimum(t, last), 0)),
            pl.BlockSpec((e, d), lambda t: (0, 0)),
            pl.BlockSpec((e, d), lambda t: (0, 0)),
        ],
        out_specs=[
            pl.BlockSpec((_TT, 2), lambda t: (jnp.minimum(t, last), 0)),
            pl.BlockSpec((_TT, 2), lambda t: (jnp.minimum(t, last), 0)),
            pl.BlockSpec((_TT, 2), lambda t: (jnp.minimum(t, last), 0)),
            pl.BlockSpec((1, e), lambda t: (0, 0)),
            pl.BlockSpec((1, n_pad), lambda t: (0, 0)),
        ],
        out_shape=[
            jax.ShapeDtypeStruct((n, 2), jnp.int32),
            jax.ShapeDtypeStruct((n, 2), jnp.float32),
            jax.ShapeDtypeStruct((n, 2), jnp.int32),
            jax.ShapeDtypeStruct((1, e), jnp.int32),
            jax.ShapeDtypeStruct((1, n_pad), jnp.int32),
        ],
        scratch_shapes=[pltpu.VMEM((1, e), jnp.float32)],
    )(x_gate, noise, Wg, Wn)


_RC = 32


def _dispatch_body(x_hbm, idx_hbm, gr_hbm, seg_hbm, w_hbm,
                   xs_hbm, slot_hbm, wslot_hbm,
                   seg_v, idxc_v, grc_v, slot_v, slot2d_v, tok_v, wc_v,
                   rows_v, *sems):
    cid = lax.axis_index("c")
    sid = lax.axis_index("s")
    n_asg = idx_hbm.shape[0]
    a_chunk = n_asg // 32
    wid = sid * 2 + cid
    base = wid * a_chunk

    pltpu.sync_copy(seg_hbm, seg_v.at[pl.ds(0, 8)])
    pltpu.sync_copy(idx_hbm.at[pl.ds(base, a_chunk)], idxc_v)
    pltpu.sync_copy(gr_hbm.at[pl.ds(base, a_chunk)], grc_v)
    pltpu.sync_copy(w_hbm.at[pl.ds(base, a_chunk)], wc_v)

    segvec = seg_v[pl.ds(0, 16)]
    for i in range(a_chunk // 16):
        e_v = idxc_v[pl.ds(i * 16, 16)]
        g_v = grc_v[pl.ds(i * 16, 16)]
        s_v = g_v
        for ex in range(8):
            s_v = s_v + jnp.where(e_v == ex, segvec[ex], 0)
        slot_v[pl.ds(i * 16, 16)] = s_v
        slot2d_v[i * 16 // _RC, pl.ds(i * 16 % _RC, 16)] = s_v
        t_v = (lax.iota(jnp.int32, 16) + (base + i * 16)) >> 1
        tok_v[pl.ds(i * 16, 16)] = t_v

    pltpu.sync_copy(slot_v, slot_hbm.at[pl.ds(base, a_chunk)])
    wsc = pltpu.async_copy(wc_v, wslot_hbm.at[slot_v], sems[4])

    n_ch = a_chunk // _RC
    scat = [None, None]
    for c in range(n_ch):
        b = c % 2
        if scat[b] is not None:
            scat[b].wait()
        g = pltpu.async_copy(x_hbm.at[tok_v.at[pl.ds(c * _RC, _RC)]],
                             rows_v.at[b], sems[b])
        g.wait()
        scat[b] = pltpu.async_copy(rows_v.at[b], xs_hbm.at[slot2d_v.at[c]],
                                   sems[2 + b])
    scat[0].wait()
    scat[1].wait()
    wsc.wait()


def _dispatch(x_experts, top_idx, gr, seg, top_w, cap):
    n, d = x_experts.shape
    n_asg = 2 * n
    mesh = plsc.VectorSubcoreMesh(core_axis_name="c", subcore_axis_name="s")
    a_chunk = n_asg // 32
    kfn = functools.partial(
        pl.kernel,
        mesh=mesh,
        out_type=[
            jax.ShapeDtypeStruct((cap, d), jnp.float32),
            jax.ShapeDtypeStruct((n_asg,), jnp.int32),
            jax.ShapeDtypeStruct((cap,), jnp.float32),
        ],
        scratch_types=[
            pltpu.VMEM((16,), jnp.int32),
            pltpu.VMEM((a_chunk,), jnp.int32),
            pltpu.VMEM((a_chunk,), jnp.int32),
            pltpu.VMEM((a_chunk,), jnp.int32),
            pltpu.VMEM((a_chunk // _RC, _RC), jnp.int32),
            pltpu.VMEM((a_chunk,), jnp.int32),
            pltpu.VMEM((a_chunk,), jnp.float32),
            pltpu.VMEM((2, _RC, d), jnp.float32),
            pltpu.SemaphoreType.DMA,
            pltpu.SemaphoreType.DMA,
            pltpu.SemaphoreType.DMA,
            pltpu.SemaphoreType.DMA,
            pltpu.SemaphoreType.DMA,
        ],
    )
    return kfn(_dispatch_body)(x_experts, top_idx, gr, seg, top_w)


def _gmm_body(gmap_ref, xs_ref, we_ref, be_ref, ws_ref, ys_ref):
    t = pl.program_id(0)
    n_used = gmap_ref[gmap_ref.shape[0] - 8]

    @pl.when(t < n_used)
    def _():
        x = xs_ref[...].astype(jnp.float32)
        y = jax.lax.dot_general(x, we_ref[0], (((1,), (1,)), ((), ())),
                                preferred_element_type=jnp.float32)
        ys_ref[...] = (y + be_ref[0]) * ws_ref[...]


def _gmm(xs, We, be3, wslot2, gmap, n_tiles):
    cap, d = xs.shape
    n_pad = gmap.shape[0]

    def _tc(t, gm):
        return jnp.minimum(t, gm[n_pad - 8] - 1)

    grid_spec = pltpu.PrefetchScalarGridSpec(
        num_scalar_prefetch=1,
        grid=(n_tiles,),
        in_specs=[
            pl.BlockSpec((_T, d), lambda t, gm: (_tc(t, gm), 0)),
            pl.BlockSpec((1, d, d), lambda t, gm: (gm[_tc(t, gm)], 0, 0)),
            pl.BlockSpec((1, 1, d), lambda t, gm: (gm[_tc(t, gm)], 0, 0)),
            pl.BlockSpec((_T, 1), lambda t, gm: (_tc(t, gm), 0)),
        ],
        out_specs=pl.BlockSpec((_T, d), lambda t, gm: (_tc(t, gm), 0)),
    )
    return pl.pallas_call(
        _gmm_body,
        grid_spec=grid_spec,
        out_shape=jax.ShapeDtypeStruct((cap, d), jnp.float32),
    )(gmap, xs, We, be3, wslot2)


_TCK = 16


def _combine_body(ys_hbm, s0_hbm, s1_hbm, out_hbm,
                  s0_v, s1_v, rows_v, out_v, *sems):
    cid = lax.axis_index("c")
    sid = lax.axis_index("s")
    wid = sid * 2 + cid
    n_tok = out_hbm.shape[0]
    d = out_hbm.shape[1]
    t_share = n_tok // 32
    t0 = wid * t_share
    pltpu.sync_copy(s0_hbm.at[pl.ds(t0, t_share)], s0_v)
    pltpu.sync_copy(s1_hbm.at[pl.ds(t0, t_share)], s1_v)

    n_ch = t_share // _TCK
    geven = [None, None]
    godd = [None, None]
    ostore = [None, None]

    def _issue(c):
        b = c % 2
        if ostore[b] is not None:
            ostore[b].wait()
        geven[b] = pltpu.async_copy(
            ys_hbm.at[s0_v.at[pl.ds(c * _TCK, _TCK)]], out_v.at[b],
            sems[b])
        godd[b] = pltpu.async_copy(
            ys_hbm.at[s1_v.at[pl.ds(c * _TCK, _TCK)]], rows_v.at[b],
            sems[2 + b])

    _issue(0)
    for c in range(n_ch):
        b = c % 2
        geven[b].wait()
        godd[b].wait()
        if c + 1 < n_ch:
            _issue(c + 1)

        def _tok(i, _, b=b):

            def _lane(j, _):
                r1 = rows_v[b, i, pl.ds(j * 16, 16)]
                plsc.addupdate(out_v.at[b, i, pl.ds(j * 16, 16)], r1)
                return ()

            lax.fori_loop(0, d // 16, _lane, (), unroll=8)
            return ()

        lax.fori_loop(0, _TCK, _tok, (), unroll=1)

        ostore[b] = pltpu.async_copy(
            out_v.at[b], out_hbm.at[pl.ds(t0 + c * _TCK, _TCK)], sems[4 + b])
    ostore[0].wait()
    ostore[1].wait()


def _combine(ys, s0, s1, n, d):
    mesh = plsc.VectorSubcoreMesh(core_axis_name="c", subcore_axis_name="s")
    t_share = n // 32
    kfn = functools.partial(
        pl.kernel,
        mesh=mesh,
        out_type=jax.ShapeDtypeStruct((n, d), jnp.float32),
        scratch_types=[
            pltpu.VMEM((t_share,), jnp.int32),
            pltpu.VMEM((t_share,), jnp.int32),
            pltpu.VMEM((2, _TCK, d), jnp.float32),
            pltpu.VMEM((2, _TCK, d), jnp.float32),
            pltpu.SemaphoreType.DMA,
            pltpu.SemaphoreType.DMA,
            pltpu.SemaphoreType.DMA,
            pltpu.SemaphoreType.DMA,
            pltpu.SemaphoreType.DMA,
            pltpu.SemaphoreType.DMA,
        ],
    )
    return kfn(_combine_body)(ys, s0, s1)


def kernel(x_gate, x_experts, noise, Wg, Wn, We, be):
    n, d = x_gate.shape
    e = Wg.shape[0]
    cap = 2 * n + e * _T
    n_tiles = cap // _T

    n_pad = n_tiles + 8
    top_idx, top_w, gr, seg, gmap = _gating(x_gate, noise, Wg, Wn, n_pad)
    xs, slot, wslot = _dispatch(x_experts, top_idx.reshape(-1),
                                gr.reshape(-1), seg.reshape(-1),
                                top_w.reshape(-1), cap)
    ys = _gmm(xs, We, be.reshape(e, 1, d), wslot.reshape(cap, 1),
              gmap.reshape(-1), n_tiles)
    slot2 = slot.reshape(n, 2)
    return _combine(ys, slot2[:, 0], slot2[:, 1], n, d)


# speedup vs baseline: 1.2099x; 1.2099x over previous
import functools

import jax
import jax.numpy as jnp
from jax import lax
from jax.experimental import pallas as pl
from jax.experimental.pallas import tpu as pltpu
from jax.experimental.pallas import tpu_sc as plsc

_NEG = -1e30
_T = 256
_TT = 1024


def _gate_body(xg_ref, nz_ref, wg_ref, wn_ref,
               idx_ref, w_ref, gr_ref, seg_ref, gmap_ref, cnt_ref):
    t = pl.program_id(0)
    nt = pl.num_programs(0) - 1
    n_exp = wg_ref.shape[0]

    @pl.when(t == 0)
    def _init():
        cnt_ref[...] = jnp.zeros_like(cnt_ref)

    @pl.when(t < nt)
    def _tile():
        xg = xg_ref[...]
        dn = (((1,), (1,)), ((), ()))
        clean = jax.lax.dot_general(xg, wg_ref[...], dn,
                                    preferred_element_type=jnp.float32)
        raw = jax.lax.dot_general(xg, wn_ref[...], dn,
                                  preferred_element_type=jnp.float32)
        sp = jnp.maximum(raw, 0.0) + jnp.log1p(jnp.exp(-jnp.abs(raw)))
        noisy = clean + nz_ref[...] * sp
        cols = jax.lax.broadcasted_iota(jnp.int32, noisy.shape, 1)
        m1 = jnp.max(noisy, axis=1, keepdims=True)
        i1 = jnp.min(jnp.where(noisy == m1, cols, n_exp), axis=1,
                     keepdims=True)
        oh1 = (cols == i1).astype(jnp.float32)
        masked = jnp.where(cols == i1, _NEG, noisy)
        m2 = jnp.max(masked, axis=1, keepdims=True)
        i2 = jnp.min(jnp.where(masked == m2, cols, n_exp), axis=1,
                     keepdims=True)
        oh2 = (cols == i2).astype(jnp.float32)
        z = jnp.exp(m2 - m1)
        w1 = 1.0 / (1.0 + z)
        w2 = z / (1.0 + z)

        idx_ref[...] = jnp.concatenate(
            [i1.astype(jnp.int32), i2.astype(jnp.int32)], axis=1)
        w_ref[...] = jnp.concatenate([w1, w2], axis=1)

        ohs = oh1 + oh2
        r = jax.lax.broadcasted_iota(jnp.int32, (ohs.shape[0],) * 2, 0)
        c = jax.lax.broadcasted_iota(jnp.int32, (ohs.shape[0],) * 2, 1)
        tril = (r > c).astype(jnp.float32)
        prior = jax.lax.dot_general(tril, ohs, (((1,), (0,)), ((), ())),
                                    preferred_element_type=jnp.float32)
        carry = cnt_ref[...]
        g1 = jnp.sum((prior + carry) * oh1, axis=1, keepdims=True)
        g2 = jnp.sum((prior + carry) * oh2, axis=1, keepdims=True)
        gr_ref[...] = jnp.concatenate(
            [g1.astype(jnp.int32), g2.astype(jnp.int32)], axis=1)
        cnt_ref[...] = carry + jnp.sum(ohs, axis=0, keepdims=True)

    @pl.when(t == nt)
    def _finalize():
        counts = cnt_ref[...]
        seg_size = jnp.ceil(counts / _T) * _T
        r = jax.lax.broadcasted_iota(jnp.int32, (n_exp, n_exp), 0)
        c = jax.lax.broadcasted_iota(jnp.int32, (n_exp, n_exp), 1)
        excl = (r < c).astype(jnp.float32)
        seg_start = jax.lax.dot_general(
            seg_size, excl, (((1,), (0,)), ((), ())),
            preferred_element_type=jnp.float32)
        ends = seg_start + seg_size
        seg_ref[...] = seg_start.astype(jnp.int32)
        n_pad = gmap_ref.shape[1]
        jt = jax.lax.broadcasted_iota(
            jnp.int32, (n_pad, n_exp), 0).astype(jnp.float32) * _T
        gm = jnp.sum((jt >= ends).astype(jnp.int32), axis=1)
        gm = jnp.minimum(gm, n_exp - 1)
        n_used = (jnp.sum(seg_size) / _T).astype(jnp.int32)
        jcol = jax.lax.broadcasted_iota(jnp.int32, (n_pad,), 0)
        gm = jnp.where(jcol == n_pad - 8, n_used, gm)
        gmap_ref[...] = gm.reshape(1, n_pad)


def _gating(x_gate, noise, Wg, Wn, n_pad):
    n, d = x_gate.shape
    e = Wg.shape[0]
    nt = n // _TT
    last = nt - 1
    return pl.pallas_call(
        _gate_body,
        grid=(nt + 1,),
        in_specs=[
            pl.BlockSpec((_TT, d), lambda t: (jnp.minimum(t, last), 0)),
            pl.BlockSpec((_TT, e), lambda t: (jnp.minimum(t, last), 0)),
            pl.BlockSpec((e, d), lambda t: (0, 0)),
            pl.BlockSpec((e, d), lambda t: (0, 0)),
        ],
        out_specs=[
            pl.BlockSpec((_TT, 2), lambda t: (jnp.minimum(t, last), 0)),
            pl.BlockSpec((_TT, 2), lambda t: (jnp.minimum(t, last), 0)),
            pl.BlockSpec((_TT, 2), lambda t: (jnp.minimum(t, last), 0)),
            pl.BlockSpec((1, e), lambda t: (0, 0)),
            pl.BlockSpec((1, n_pad), lambda t: (0, 0)),
        ],
        out_shape=[
            jax.ShapeDtypeStruct((n, 2), jnp.int32),
            jax.ShapeDtypeStruct((n, 2), jnp.float32),
            jax.ShapeDtypeStruct((n, 2), jnp.int32),
            jax.ShapeDtypeStruct((1, e), jnp.int32),
            jax.ShapeDtypeStruct((1, n_pad), jnp.int32),
        ],
        scratch_shapes=[pltpu.VMEM((1, e), jnp.float32)],
    )(x_gate, noise, Wg, Wn)


_RC = 32


def _dispatch_body(x_hbm, idx_hbm, gr_hbm, seg_hbm, xs_hbm, slot_hbm,
                   seg_v, idxc_v, grc_v, slot_v, slot2d_v, tok_v, rows_v,
                   *sems):
    cid = lax.axis_index("c")
    sid = lax.axis_index("s")
    n_asg = idx_hbm.shape[0]
    a_chunk = n_asg // 32
    wid = sid * 2 + cid
    base = wid * a_chunk

    pltpu.sync_copy(seg_hbm, seg_v.at[pl.ds(0, 8)])
    pltpu.sync_copy(idx_hbm.at[pl.ds(base, a_chunk)], idxc_v)
    pltpu.sync_copy(gr_hbm.at[pl.ds(base, a_chunk)], grc_v)

    segvec = seg_v[pl.ds(0, 16)]
    for i in range(a_chunk // 16):
        e_v = idxc_v[pl.ds(i * 16, 16)]
        g_v = grc_v[pl.ds(i * 16, 16)]
        s_v = g_v
        for ex in range(8):
            s_v = s_v + jnp.where(e_v == ex, segvec[ex], 0)
        slot_v[pl.ds(i * 16, 16)] = s_v
        slot2d_v[i * 16 // _RC, pl.ds(i * 16 % _RC, 16)] = s_v
        t_v = (lax.iota(jnp.int32, 16) + (base + i * 16)) >> 1
        tok_v[pl.ds(i * 16, 16)] = t_v

    pltpu.sync_copy(slot_v, slot_hbm.at[pl.ds(base, a_chunk)])

    n_ch = a_chunk // _RC
    scat = [None, None]
    for c in range(n_ch):
        b = c % 2
        if scat[b] is not None:
            scat[b].wait()
        g = pltpu.async_copy(x_hbm.at[tok_v.at[pl.ds(c * _RC, _RC)]],
                             rows_v.at[b], sems[b])
        g.wait()
        scat[b] = pltpu.async_copy(rows_v.at[b], xs_hbm.at[slot2d_v.at[c]],
                                   sems[2 + b])
    scat[0].wait()
    scat[1].wait()


def _dispatch(x_experts, top_idx, gr, seg, cap):
    n, d = x_experts.shape
    n_asg = 2 * n
    mesh = plsc.VectorSubcoreMesh(core_axis_name="c", subcore_axis_name="s")
    a_chunk = n_asg // 32
    kfn = functools.partial(
        pl.kernel,
        mesh=mesh,
        out_type=[
            jax.ShapeDtypeStruct((cap, d), jnp.float32),
            jax.ShapeDtypeStruct((n_asg,), jnp.int32),
        ],
        scratch_types=[
            pltpu.VMEM((16,), jnp.int32),
            pltpu.VMEM((a_chunk,), jnp.int32),
            pltpu.VMEM((a_chunk,), jnp.int32),
            pltpu.VMEM((a_chunk,), jnp.int32),
            pltpu.VMEM((a_chunk // _RC, _RC), jnp.int32),
            pltpu.VMEM((a_chunk,), jnp.int32),
            pltpu.VMEM((2, _RC, d), jnp.float32),
            pltpu.SemaphoreType.DMA,
            pltpu.SemaphoreType.DMA,
            pltpu.SemaphoreType.DMA,
            pltpu.SemaphoreType.DMA,
        ],
    )
    return kfn(_dispatch_body)(x_experts, top_idx, gr, seg)


def _gmm_body(gmap_ref, xs_ref, we_ref, be_ref, ys_ref):
    t = pl.program_id(0)
    n_used = gmap_ref[gmap_ref.shape[0] - 8]

    @pl.when(t < n_used)
    def _():
        x = xs_ref[...].astype(jnp.float32)
        y = jax.lax.dot_general(x, we_ref[0], (((1,), (1,)), ((), ())),
                                preferred_element_type=jnp.float32)
        ys_ref[...] = y + be_ref[0]


def _gmm(xs, We, be3, gmap, n_tiles):
    cap, d = xs.shape
    n_pad = gmap.shape[0]

    def _tc(t, gm):
        return jnp.minimum(t, gm[n_pad - 8] - 1)

    grid_spec = pltpu.PrefetchScalarGridSpec(
        num_scalar_prefetch=1,
        grid=(n_tiles,),
        in_specs=[
            pl.BlockSpec((_T, d), lambda t, gm: (_tc(t, gm), 0)),
            pl.BlockSpec((1, d, d), lambda t, gm: (gm[_tc(t, gm)], 0, 0)),
            pl.BlockSpec((1, 1, d), lambda t, gm: (gm[_tc(t, gm)], 0, 0)),
        ],
        out_specs=pl.BlockSpec((_T, d), lambda t, gm: (_tc(t, gm), 0)),
    )
    return pl.pallas_call(
        _gmm_body,
        grid_spec=grid_spec,
        out_shape=jax.ShapeDtypeStruct((cap, d), jnp.float32),
    )(gmap, xs, We, be3)


_TCK = 16


def _combine_body(ys_hbm, slot_hbm, w_hbm, out_hbm,
                  slot_v, w_v, rows_v, out_v, sem0, sem1, semo0, semo1):
    cid = lax.axis_index("c")
    sid = lax.axis_index("s")
    wid = sid * 2 + cid
    n_tok = out_hbm.shape[0]
    d = out_hbm.shape[1]
    t_share = n_tok // 32
    t0 = wid * t_share
    pltpu.sync_copy(slot_hbm.at[pl.ds(t0 * 2, t_share * 2)], slot_v)
    pltpu.sync_copy(w_hbm.at[pl.ds(t0 * 2, t_share * 2)],
                    w_v.at[pl.ds(0, t_share * 2)])

    sems = [sem0, sem1]
    osems = [semo0, semo1]
    n_ch = t_share // _TCK
    gath = [None, None]
    ostore = [None, None]

    def _issue(c):
        b = c % 2
        idx = slot_v.at[pl.ds(c * _TCK * 2, _TCK * 2)]
        gath[b] = pltpu.async_copy(ys_hbm.at[idx], rows_v.at[b], sems[b])

    _issue(0)
    for c in range(n_ch):
        b = c % 2
        gath[b].wait()
        if c + 1 < n_ch:
            _issue(c + 1)
        if ostore[b] is not None:
            ostore[b].wait()

        for i in range(_TCK):
            pair = w_v[pl.ds(c * _TCK * 2 + 2 * i, 16)]
            w0 = pair[0]
            w1 = pair[1]

            def _lane(j, _, b=b, i=i, w0=w0, w1=w1):
                r0 = rows_v[b, 2 * i, pl.ds(j * 16, 16)]
                r1 = rows_v[b, 2 * i + 1, pl.ds(j * 16, 16)]
                out_v[b, i, pl.ds(j * 16, 16)] = w0 * r0 + w1 * r1
                return ()

            lax.fori_loop(0, d // 16, _lane, (), unroll=8)

        ostore[b] = pltpu.async_copy(
            out_v.at[b], out_hbm.at[pl.ds(t0 + c * _TCK, _TCK)], osems[b])
    ostore[0].wait()
    ostore[1].wait()


def _combine(ys, slot, top_w, n, d):
    mesh = plsc.VectorSubcoreMesh(core_axis_name="c", subcore_axis_name="s")
    t_share = n // 32
    kfn = functools.partial(
        pl.kernel,
        mesh=mesh,
        out_type=jax.ShapeDtypeStruct((n, d), jnp.float32),
        scratch_types=[
            pltpu.VMEM((t_share * 2,), jnp.int32),
            pltpu.VMEM((t_share * 2 + 16,), jnp.float32),
            pltpu.VMEM((2, _TCK * 2, d), jnp.float32),
            pltpu.VMEM((2, _TCK, d), jnp.float32),
            pltpu.SemaphoreType.DMA,
            pltpu.SemaphoreType.DMA,
            pltpu.SemaphoreType.DMA,
            pltpu.SemaphoreType.DMA,
        ],
    )
    return kfn(_combine_body)(ys, slot, top_w)


def kernel(x_gate, x_experts, noise, Wg, Wn, We, be):
    n, d = x_gate.shape
    e = Wg.shape[0]
    cap = 2 * n + e * _T
    n_tiles = cap // _T

    n_pad = n_tiles + 8
    top_idx, top_w, gr, seg, gmap = _gating(x_gate, noise, Wg, Wn, n_pad)
    xs, slot = _dispatch(x_experts, top_idx.reshape(-1),
                         gr.reshape(-1), seg.reshape(-1), cap)
    ys = _gmm(xs, We, be.reshape(e, 1, d), gmap.reshape(-1), n_tiles)
    return _combine(ys, slot, top_w.reshape(-1), n, d)
